# Initial kernel scaffold; baseline (speedup 1.0000x reference)
import jax
import jax.numpy as jnp
from jax.experimental import pallas as pl


def kernel(decoder_hidden_state, batch_decoder_input_hh, batch_decoder_input_attender, W_hh, W_map, sparse_b, sparse_t, sparse_val):
    raise NotImplementedError("write your pallas kernel here")


# speedup vs baseline: 14.7682x; 14.7682x over previous
import functools

import jax
import jax.numpy as jnp
from jax import lax
from jax.experimental import pallas as pl
from jax.experimental.pallas import tpu as pltpu
from jax.experimental.pallas import tpu_sc as plsc

B, T, V, H, D = 64, 2048, 512, 512, 512
NNZ = 16384
TBITS = 11
NC, NS = 2, 16
NW = NC * NS
BPW = B // NW
L = 16
CH = 16
VCH = V // L
RSPAD = 80
NEG = -3.0e38


def _g_tc_kernel(dhs_ref, wmap_ref, whh_ref, coef_ref, g_ref):
    w2 = jnp.dot(wmap_ref[...], whh_ref[...],
                 preferred_element_type=jnp.float32)
    g_ref[...] = coef_ref[0, 0] * jnp.sum(
        dhs_ref[...] * w2, axis=1, keepdims=True)


def _exp16(x):
    x = jnp.maximum(x, -87.0)
    y = x * 1.4426950408889634
    kf = (y + 12582912.0) - 12582912.0
    z = x - kf * 0.6931471805599453
    p = 1.0 / 720.0
    p = p * z + 1.0 / 120.0
    p = p * z + 1.0 / 24.0
    p = p * z + 1.0 / 6.0
    p = p * z + 0.5
    p = p * z + 1.0
    p = p * z + 1.0
    ki = kf.astype(jnp.int32)
    scale = plsc.bitcast((ki + 127) << 23, jnp.float32)
    return p * scale


def _sc_body(hh_hbm, att_hbm, ks_hbm, u_hbm, rs_hbm, g_hbm, wm_hbm,
             alpha_hbm, c_hbm,
             ks_v, u_v, rs_v, g_v, wm_v, s_v, rows_v, part_v,
             arow_v, crow_v, sem):
    wid = lax.axis_index("s") * NC + lax.axis_index("c")
    lane = lax.iota(jnp.int32, 16)

    pltpu.sync_copy(ks_hbm, ks_v)
    pltpu.sync_copy(u_hbm, u_v)
    pltpu.sync_copy(rs_hbm, rs_v)
    pltpu.sync_copy(g_hbm, g_v)
    pltpu.sync_copy(wm_hbm, wm_v)

    def b_body(jj, _):
        b = wid * BPW + jj
        bsp = jnp.full((L,), b, jnp.int32)
        start = jnp.max(plsc.load_gather(rs_v, [bsp]))
        end = jnp.max(plsc.load_gather(rs_v, [bsp + 1]))
        n = end - start
        nch = (n + CH - 1) // CH
        gb = plsc.load_gather(g_v, [bsp])

        def p1(i, mcur):
            sp_pos = i * CH + lane
            posc = jnp.minimum(start + sp_pos, NNZ - 1)
            kk = plsc.load_gather(ks_v, [posc])
            pltpu.async_copy(hh_hbm.at[kk], rows_v, sem).wait()
            wmr = [wm_v[pl.ds(L * j, L)] for j in range(VCH)]
            for r in range(CH):
                acc = rows_v[r, pl.ds(0, L)] * wmr[0]
                for j in range(1, VCH):
                    acc = acc + rows_v[r, pl.ds(L * j, L)] * wmr[j]
                part_v[pl.ds(L * r, L)] = acc
            s16 = plsc.load_gather(part_v, [lane * L])
            for l in range(1, L):
                s16 = s16 + plsc.load_gather(part_v, [lane * L + l])
            s16 = s16 + gb
            plsc.store_scatter(s_v, [sp_pos], s16)
            valid = sp_pos < n
            return jnp.maximum(mcur, jnp.where(valid, s16, NEG))

        mvec = lax.fori_loop(0, nch, p1, jnp.full((L,), NEG, jnp.float32))
        msp = jnp.full((L,), jnp.max(mvec), jnp.float32)

        def p2(i, dcur):
            sp_pos = i * CH + lane
            sv = plsc.load_gather(s_v, [sp_pos])
            posc = jnp.minimum(start + sp_pos, NNZ - 1)
            uu = plsc.load_gather(u_v, [posc])
            valid = sp_pos < n
            e = _exp16(jnp.where(valid, sv - msp, 0.0))
            return dcur + e * jnp.where(valid, uu, 0.0)

        dvec = lax.fori_loop(0, nch, p2, jnp.zeros((L,), jnp.float32))
        den = jnp.full((L,), jnp.sum(dvec), jnp.float32)
        rsp = jnp.ones((L,), jnp.float32) / jnp.maximum(den, 1e-30)

        zz = jnp.zeros((L,), jnp.float32)
        for j in range(T // L):
            arow_v[pl.ds(L * j, L)] = zz

        zz = jnp.zeros((L,), jnp.float32)
        for j in range(VCH):
            crow_v[pl.ds(L * j, L)] = zz

        def p3(i, _):
            sp_pos = i * CH + lane
            posc = jnp.minimum(start + sp_pos, NNZ - 1)
            kk = plsc.load_gather(ks_v, [posc])
            sv = plsc.load_gather(s_v, [sp_pos])
            uu = plsc.load_gather(u_v, [posc])
            valid = sp_pos < n
            e = _exp16(jnp.where(valid, sv - msp, 0.0))
            al = e * rsp
            plsc.store_scatter(arow_v, [kk & (T - 1)], al, mask=valid)
            wv = jnp.where(valid, uu, 0.0) * al
            pltpu.async_copy(att_hbm.at[kk], rows_v, sem).wait()
            for r in range(CH):
                wr = jnp.full(
                    (L,), jnp.sum(jnp.where(lane == r, wv, 0.0)), jnp.float32)
                for j in range(VCH):
                    crow_v[pl.ds(L * j, L)] = (
                        crow_v[pl.ds(L * j, L)]
                        + rows_v[r, pl.ds(L * j, L)] * wr)
            return 0

        lax.fori_loop(0, nch, p3, 0)
        pltpu.sync_copy(arow_v, alpha_hbm.at[b])
        pltpu.sync_copy(crow_v, c_hbm.at[b])
        return 0

    lax.fori_loop(0, BPW, b_body, 0)


_sc_call = functools.partial(
    pl.kernel,
    mesh=plsc.VectorSubcoreMesh(core_axis_name="c", subcore_axis_name="s"),
    compiler_params=pltpu.CompilerParams(needs_layout_passes=False),
    out_type=[jax.ShapeDtypeStruct((B, T), jnp.float32),
              jax.ShapeDtypeStruct((B, D), jnp.float32)],
    scratch_types=[
        pltpu.VMEM((NNZ,), jnp.int32),
        pltpu.VMEM((NNZ,), jnp.float32),
        pltpu.VMEM((RSPAD,), jnp.int32),
        pltpu.VMEM((B,), jnp.float32),
        pltpu.VMEM((V,), jnp.float32),
        pltpu.VMEM((NNZ,), jnp.float32),
        pltpu.VMEM((CH, V), jnp.float32),
        pltpu.VMEM((CH * L,), jnp.float32),
        pltpu.VMEM((T,), jnp.float32),
        pltpu.VMEM((D,), jnp.float32),
        pltpu.SemaphoreType.DMA,
    ],
)(_sc_body)


def kernel(decoder_hidden_state, batch_decoder_input_hh,
           batch_decoder_input_attender, W_hh, W_map,
           sparse_b, sparse_t, sparse_val):
    f32 = jnp.float32
    sb = sparse_b.astype(jnp.int32)
    st = sparse_t.astype(jnp.int32)
    ks = jnp.sort(sb * T + st)
    u = jnp.concatenate(
        [jnp.ones((1,), f32), (ks[1:] != ks[:-1]).astype(f32)])
    rs = jnp.searchsorted(ks >> TBITS,
                          jnp.arange(B + 1, dtype=jnp.int32)).astype(jnp.int32)
    rs = jnp.concatenate([rs, jnp.zeros((RSPAD - B - 1,), jnp.int32)])
    coef = jnp.where(sparse_val == 1, 1.0, 2.0).astype(f32)

    g = pl.pallas_call(
        _g_tc_kernel,
        out_shape=jax.ShapeDtypeStruct((B, 1), f32),
    )(decoder_hidden_state, W_map, W_hh, coef.reshape(1, 1))

    alpha2d, c = _sc_call(
        batch_decoder_input_hh.reshape(B * T, V),
        batch_decoder_input_attender.reshape(B * T, D),
        ks, u, rs, g[:, 0], W_map.reshape(V))
    return (c, alpha2d[:, :, None])
